# baseline (speedup 1.0000x reference)
import functools

import jax
import jax.numpy as jnp
from jax import lax
from jax.experimental import pallas as pl
from jax.experimental.pallas import tpu as pltpu
from jax.experimental.pallas import tpu_sc as plsc

N = 50000
E = 800000
DIN = 67
DH = 128
WP = 80
WH = 40
W2P = 16
EPS = 1e-5
SLOPE = 0.01

NC = 2
NS = 16
NPC = N // NC
RPT = 1568
RS = RPT * NS
CHUNK = 128
NB = 8
GRP = 49
CPT = GRP * NB
EPT = CPT * CHUNK
EP = EPT * NS

BS = 2000
GRID = N // BS


def _sc_mesh():
    return plsc.VectorSubcoreMesh(
        core_axis_name="c", subcore_axis_name="s", num_cores=NC,
        num_subcores=NS)


SB = 112
NSB = RPT // SB


def _zero_wbuf(wbuf, W):
    offs = list(range(0, W - 15, 16))
    if W % 16:
        offs.append(W - 16)

    def zrow(r, carry):
        for o in offs:
            wbuf[r, pl.ds(o, 16)] = jnp.zeros((16,), jnp.float32)
        return carry
    lax.fori_loop(0, SB, zrow, 0)


def _zero_slab_stripe(wbuf, slab, tid):
    def zcp(k, carry):
        off = tid * RPT + k * SB
        pltpu.sync_copy(wbuf, slab.at[pl.ds(off, SB)])
        return carry
    lax.fori_loop(0, NSB, zcp, 0)


def _writeout_stripe(wbuf, slab, out_hbm, cid, tid):
    def wcp(k, carry):
        off = tid * RPT + k * SB
        pltpu.sync_copy(slab.at[pl.ds(off, SB)], wbuf)
        pltpu.sync_copy(wbuf, out_hbm.at[pl.ds(cid * RS + off, SB)])
        return carry
    lax.fori_loop(0, NSB, wcp, 0)


def _local_dst_group(dstbuf, base_node):
    for j in range(NB * CHUNK // 16):
        d = dstbuf[pl.ds(j * 16, 16)]
        ld = d - base_node
        ok = (ld >= 0) & (ld < NPC)
        dstbuf[pl.ds(j * 16, 16)] = jnp.where(ok, ld, NPC)


def _hop_body(W, v_hbm, src_hbm, dst_hbm, out_hbm,
              srcbuf, dstbuf, rowbufs, wbuf, slab,
              isem, gsem, ssem):
    cid = lax.axis_index("c")
    tid = lax.axis_index("s")
    base_node = cid * NPC
    _zero_wbuf(wbuf, W)
    _zero_slab_stripe(wbuf, slab, tid)
    plsc.subcore_barrier()

    base_row = tid * CPT

    def group(g, carry):
        eb = (base_row + g * NB) * CHUNK
        ia = pltpu.async_copy(src_hbm.at[pl.ds(eb, NB * CHUNK)], srcbuf, isem)
        ib = pltpu.async_copy(dst_hbm.at[pl.ds(eb, NB * CHUNK)], dstbuf, isem)
        ia.wait()
        ib.wait()
        _local_dst_group(dstbuf, base_node)
        pltpu.async_copy(v_hbm.at[srcbuf], rowbufs, gsem).wait()
        pltpu.async_copy(rowbufs, slab.at[dstbuf], ssem, add=True).wait()
        return carry

    lax.fori_loop(0, GRP, group, 0)
    plsc.subcore_barrier()
    _writeout_stripe(wbuf, slab, out_hbm, cid, tid)


def _make_hop(W):
    body = functools.partial(_hop_body, W)
    return pl.kernel(
        body,
        out_type=jax.ShapeDtypeStruct((NC * RS, W), jnp.float32),
        mesh=_sc_mesh(),
        compiler_params=pltpu.CompilerParams(use_tc_tiling_on_sc=False),
        scratch_types=[
            pltpu.VMEM((NB * CHUNK,), jnp.int32),
            pltpu.VMEM((NB * CHUNK,), jnp.int32),
            pltpu.VMEM((NB * CHUNK, W), jnp.float32),
            pltpu.VMEM((SB, W), jnp.float32),
            pltpu.VMEM_SHARED((RS, W), jnp.float32),
            pltpu.SemaphoreType.DMA,
            pltpu.SemaphoreType.DMA,
            pltpu.SemaphoreType.DMA,
        ],
    )


def _deg_body(dst_hbm, out_hbm, dstbuf, onesbuf, wbuf, slab,
              isem, ssem):
    cid = lax.axis_index("c")
    tid = lax.axis_index("s")
    base_node = cid * NPC
    _zero_wbuf(wbuf, W2P)

    def orow(r, carry):
        onesbuf[r, pl.ds(0, 16)] = jnp.ones((16,), jnp.float32)
        return carry
    lax.fori_loop(0, NB * CHUNK, orow, 0)
    _zero_slab_stripe(wbuf, slab, tid)
    plsc.subcore_barrier()

    base_row = tid * CPT

    def group(g, carry):
        eb = (base_row + g * NB) * CHUNK
        pltpu.async_copy(dst_hbm.at[pl.ds(eb, NB * CHUNK)], dstbuf, isem).wait()
        _local_dst_group(dstbuf, base_node)
        pltpu.async_copy(onesbuf, slab.at[dstbuf], ssem, add=True).wait()
        return carry

    lax.fori_loop(0, GRP, group, 0)
    plsc.subcore_barrier()
    _writeout_stripe(wbuf, slab, out_hbm, cid, tid)


def _make_deg():
    return pl.kernel(
        _deg_body,
        out_type=jax.ShapeDtypeStruct((NC * RS, W2P), jnp.float32),
        mesh=_sc_mesh(),
        compiler_params=pltpu.CompilerParams(use_tc_tiling_on_sc=False),
        scratch_types=[
            pltpu.VMEM((NB * CHUNK,), jnp.int32),
            pltpu.VMEM((NB * CHUNK, W2P), jnp.float32),
            pltpu.VMEM((SB, W2P), jnp.float32),
            pltpu.VMEM_SHARED((RS, W2P), jnp.float32),
            pltpu.SemaphoreType.DMA,
            pltpu.SemaphoreType.DMA,
        ],
    )


def _unpad_core(a):
    return jnp.concatenate([a[:NPC], a[RS:RS + NPC]], axis=0)


def _stats_body(x_ref, sum_ref, sumsq_ref):
    i = pl.program_id(0)

    @pl.when(i == 0)
    def _():
        sum_ref[...] = jnp.zeros_like(sum_ref)
        sumsq_ref[...] = jnp.zeros_like(sumsq_ref)

    xb = x_ref[...]
    F = xb.shape[-1]
    xr = xb.reshape(BS // 8, 8, F)
    sum_ref[...] += jnp.sum(xr, axis=0)
    sumsq_ref[...] += jnp.sum(xr * xr, axis=0)


def _stats(x):
    F = x.shape[-1]
    return pl.pallas_call(
        _stats_body,
        grid=(GRID,),
        in_specs=[pl.BlockSpec((BS, F), lambda i: (i, 0))],
        out_specs=[pl.BlockSpec((8, F), lambda i: (0, 0)),
                   pl.BlockSpec((8, F), lambda i: (0, 0))],
        out_shape=[jax.ShapeDtypeStruct((8, F), jnp.float32),
                   jax.ShapeDtypeStruct((8, F), jnp.float32)],
    )(x)


def _apply1_body(x_ref, sum_ref, sumsq_ref, g_ref, b_ref, deg_ref,
                 h0_ref, hta_ref, htb_ref, dis_ref, invdeg_ref):
    mu = jnp.sum(sum_ref[...], axis=0, keepdims=True) / N
    var = jnp.sum(sumsq_ref[...], axis=0, keepdims=True) / N - mu * mu
    rstd = lax.rsqrt(var + EPS)
    h0 = (x_ref[...] - mu) * rstd * g_ref[...] + b_ref[...]
    d = deg_ref[...]
    dis = jnp.where(d > 0, lax.rsqrt(jnp.maximum(d, 1e-12)), 0.0)
    h0_ref[...] = h0
    ht = dis * h0
    hta_ref[...] = ht[:, :WH]
    htb_ref[...] = ht[:, WH:]
    dis_ref[...] = dis
    invdeg_ref[...] = dis * dis


def _apply1(x, sums, sumsqs, gamma, beta, deg):
    return pl.pallas_call(
        _apply1_body,
        grid=(GRID,),
        in_specs=[pl.BlockSpec((BS, WP), lambda i: (i, 0)),
                  pl.BlockSpec((8, WP), lambda i: (0, 0)),
                  pl.BlockSpec((8, WP), lambda i: (0, 0)),
                  pl.BlockSpec((1, WP), lambda i: (0, 0)),
                  pl.BlockSpec((1, WP), lambda i: (0, 0)),
                  pl.BlockSpec((BS, 1), lambda i: (i, 0))],
        out_specs=[pl.BlockSpec((BS, WP), lambda i: (i, 0)),
                   pl.BlockSpec((BS, WH), lambda i: (i, 0)),
                   pl.BlockSpec((BS, WH), lambda i: (i, 0)),
                   pl.BlockSpec((BS, 1), lambda i: (i, 0)),
                   pl.BlockSpec((BS, 1), lambda i: (i, 0))],
        out_shape=[jax.ShapeDtypeStruct((N, WP), jnp.float32),
                   jax.ShapeDtypeStruct((N, WH), jnp.float32),
                   jax.ShapeDtypeStruct((N, WH), jnp.float32),
                   jax.ShapeDtypeStruct((N, 1), jnp.float32),
                   jax.ShapeDtypeStruct((N, 1), jnp.float32)],
    )(x, sums, sumsqs, gamma, beta, deg)


def _scale_body(n_in, last, *refs):
    g_refs = refs[:n_in]
    dis_ref, invdeg_ref = refs[n_in], refs[n_in + 1]
    p_refs = refs[n_in + 2:n_in + 2 + n_in]
    v_refs = refs[n_in + 2 + n_in:]
    dis = dis_ref[...]
    if not last:
        invdeg = invdeg_ref[...]
    for k in range(n_in):
        g = g_refs[k][...]
        p_refs[k][...] = dis * g
        if not last:
            v_refs[k][...] = invdeg * g


def _scale(gs, dis, invdeg, last=False):
    n_in = len(gs)
    n_out = n_in if last else 2 * n_in
    blk = [pl.BlockSpec((BS, g.shape[-1]), lambda i: (i, 0)) for g in gs]
    out = pl.pallas_call(
        functools.partial(_scale_body, n_in, last),
        grid=(GRID,),
        in_specs=blk + [pl.BlockSpec((BS, 1), lambda i: (i, 0)),
                        pl.BlockSpec((BS, 1), lambda i: (i, 0))],
        out_specs=blk * (1 if last else 2),
        out_shape=[jax.ShapeDtypeStruct((N, g.shape[-1]), jnp.float32)
                   for g in gs] * (1 if last else 2),
    )(*gs, dis, invdeg)
    if last:
        return out, None
    return out[:n_in], out[n_in:]


def _mm1_body(h0_ref, p1a_ref, p1b_ref, p2a_ref, p2b_ref, p3a_ref, p3b_ref,
              w0_ref, wa_ref, wb_ref, b_ref, out_ref, sum_ref, sumsq_ref):
    i = pl.program_id(0)
    f32 = jnp.float32
    acc = jnp.dot(h0_ref[...], w0_ref[...], preferred_element_type=f32)
    acc += jnp.dot(p1a_ref[...], wa_ref[0], preferred_element_type=f32)
    acc += jnp.dot(p1b_ref[...], wb_ref[0], preferred_element_type=f32)
    acc += jnp.dot(p2a_ref[...], wa_ref[1], preferred_element_type=f32)
    acc += jnp.dot(p2b_ref[...], wb_ref[1], preferred_element_type=f32)
    acc += jnp.dot(p3a_ref[...], wa_ref[2], preferred_element_type=f32)
    acc += jnp.dot(p3b_ref[...], wb_ref[2], preferred_element_type=f32)
    acc += b_ref[...]
    out_ref[...] = acc

    @pl.when(i == 0)
    def _():
        sum_ref[...] = jnp.zeros_like(sum_ref)
        sumsq_ref[...] = jnp.zeros_like(sumsq_ref)

    ar = acc.reshape(BS // 8, 8, DH)
    sum_ref[...] += jnp.sum(ar, axis=0)
    sumsq_ref[...] += jnp.sum(ar * ar, axis=0)


def _mm1(h0, ps, w0, wa, wb, b):
    phalf = pl.BlockSpec((BS, WH), lambda i: (i, 0))
    return pl.pallas_call(
        _mm1_body,
        grid=(GRID,),
        in_specs=[pl.BlockSpec((BS, WP), lambda i: (i, 0))]
                 + [phalf] * 6
                 + [pl.BlockSpec((WP, DH), lambda i: (0, 0)),
                    pl.BlockSpec((3, WH, DH), lambda i: (0, 0, 0)),
                    pl.BlockSpec((3, WH, DH), lambda i: (0, 0, 0)),
                    pl.BlockSpec((1, DH), lambda i: (0, 0))],
        out_specs=[pl.BlockSpec((BS, DH), lambda i: (i, 0)),
                   pl.BlockSpec((8, DH), lambda i: (0, 0)),
                   pl.BlockSpec((8, DH), lambda i: (0, 0))],
        out_shape=[jax.ShapeDtypeStruct((N, DH), jnp.float32),
                   jax.ShapeDtypeStruct((8, DH), jnp.float32),
                   jax.ShapeDtypeStruct((8, DH), jnp.float32)],
    )(h0, *ps, w0, wa, wb, b)


def _z_body(o_ref, sum_ref, sumsq_ref, g_ref, b_ref, wz_ref, dis_ref,
            z0_ref, u0_ref):
    mu = jnp.sum(sum_ref[...], axis=0, keepdims=True) / N
    var = jnp.sum(sumsq_ref[...], axis=0, keepdims=True) / N - mu * mu
    rstd = lax.rsqrt(var + EPS)
    h2 = (o_ref[...] - mu) * rstd * g_ref[...] + b_ref[...]
    h2 = jnp.where(h2 > 0, h2, SLOPE * h2)
    zf = jnp.dot(h2, wz_ref[...], preferred_element_type=jnp.float32)
    z0_ref[...] = zf[:, 0:1]
    mask = (lax.broadcasted_iota(jnp.int32, (1, W2P), 1) >= 1) & (
        lax.broadcasted_iota(jnp.int32, (1, W2P), 1) <= 3)
    u0_ref[...] = jnp.where(mask, dis_ref[...] * zf, 0.0)


def _zproj(out1, sums, sumsqs, gamma, beta, wz, dis):
    return pl.pallas_call(
        _z_body,
        grid=(GRID,),
        in_specs=[pl.BlockSpec((BS, DH), lambda i: (i, 0)),
                  pl.BlockSpec((8, DH), lambda i: (0, 0)),
                  pl.BlockSpec((8, DH), lambda i: (0, 0)),
                  pl.BlockSpec((1, DH), lambda i: (0, 0)),
                  pl.BlockSpec((1, DH), lambda i: (0, 0)),
                  pl.BlockSpec((DH, W2P), lambda i: (0, 0)),
                  pl.BlockSpec((BS, 1), lambda i: (i, 0))],
        out_specs=[pl.BlockSpec((BS, 1), lambda i: (i, 0)),
                   pl.BlockSpec((BS, W2P), lambda i: (i, 0))],
        out_shape=[jax.ShapeDtypeStruct((N, 1), jnp.float32),
                   jax.ShapeDtypeStruct((N, W2P), jnp.float32)],
    )(out1, sums, sumsqs, gamma, beta, wz, dis)


def _final_body(z0_ref, p1_ref, p2_ref, p3_ref, b_ref, out_ref):
    out_ref[...] = (z0_ref[...] + p1_ref[:, 1:2] + p2_ref[:, 2:3]
                    + p3_ref[:, 3:4] + b_ref[...])


def _final(z0, p1, p2, p3, b2):
    return pl.pallas_call(
        _final_body,
        grid=(GRID,),
        in_specs=[pl.BlockSpec((BS, 1), lambda i: (i, 0)),
                  pl.BlockSpec((BS, W2P), lambda i: (i, 0)),
                  pl.BlockSpec((BS, W2P), lambda i: (i, 0)),
                  pl.BlockSpec((BS, W2P), lambda i: (i, 0)),
                  pl.BlockSpec((1, 1), lambda i: (0, 0))],
        out_specs=pl.BlockSpec((BS, 1), lambda i: (i, 0)),
        out_shape=jax.ShapeDtypeStruct((N, 1), jnp.float32),
    )(z0, p1, p2, p3, b2)


def kernel(x, edge_index, gamma1, beta1, W1, b1, gamma2, beta2, W2, b2):
    f32 = jnp.float32
    src = jnp.pad(edge_index[0].astype(jnp.int32), (0, EP - E))
    dst = jnp.pad(edge_index[1].astype(jnp.int32), (0, EP - E),
                  constant_values=-1)

    x_pad = jnp.pad(x, ((0, 0), (0, WP - DIN)))
    g1 = jnp.pad(gamma1, (0, WP - DIN)).reshape(1, WP)
    be1 = jnp.pad(beta1, (0, WP - DIN)).reshape(1, WP)
    W1p = jnp.pad(W1, ((0, 0), (0, WP - DIN), (0, 0)))
    w0 = W1p[0]
    wa = W1p[1:, :WH, :]
    wb = W1p[1:, WH:, :]
    b1r = b1.reshape(1, DH)
    g2 = gamma2.reshape(1, DH)
    be2 = beta2.reshape(1, DH)
    wz = jnp.pad(jnp.transpose(W2[:, :, 0], (1, 0)), ((0, 0), (0, W2P - 4)))
    b2r = b2.reshape(1, 1)

    hop40 = _make_hop(WH)
    hop16 = _make_hop(W2P)
    degk = _make_deg()

    deg = _unpad_core(degk(dst))[:, 0:1]
    sums1, sumsqs1 = _stats(x_pad)
    h0, hta, htb, dis, invdeg = _apply1(x_pad, sums1, sumsqs1, g1, be1, deg)

    def hop_pair(va, vb):
        return (_unpad_core(hop40(va, src, dst)),
                _unpad_core(hop40(vb, src, dst)))

    g1a, g1b = hop_pair(hta, htb)
    (p1a, p1b), (v1a, v1b) = _scale([g1a, g1b], dis, invdeg)
    g2a, g2b = hop_pair(v1a, v1b)
    (p2a, p2b), (v2a, v2b) = _scale([g2a, g2b], dis, invdeg)
    g3a, g3b = hop_pair(v2a, v2b)
    (p3a, p3b), _ = _scale([g3a, g3b], dis, invdeg, last=True)

    out1, sums2, sumsqs2 = _mm1(
        h0, (p1a, p1b, p2a, p2b, p3a, p3b), w0, wa, wb, b1r)
    z0, u0 = _zproj(out1, sums2, sumsqs2, g2, be2, wz, dis)

    ha = _unpad_core(hop16(u0, src, dst))
    (q1,), (w1v,) = _scale([ha], dis, invdeg)
    hb = _unpad_core(hop16(w1v, src, dst))
    (q2,), (w2v,) = _scale([hb], dis, invdeg)
    hc = _unpad_core(hop16(w2v, src, dst))
    (q3,), _ = _scale([hc], dis, invdeg, last=True)

    return _final(z0, q1, q2, q3, b2r)


# speedup vs baseline: 1.0212x; 1.0212x over previous
import functools

import jax
import jax.numpy as jnp
from jax import lax
from jax.experimental import pallas as pl
from jax.experimental.pallas import tpu as pltpu
from jax.experimental.pallas import tpu_sc as plsc

N = 50000
E = 800000
DIN = 67
DH = 128
WP = 80
WH = 40
W2P = 16
EPS = 1e-5
SLOPE = 0.01

NC = 2
NS = 16
NPC = N // NC
RPT = 1568
RS = RPT * NS
CHUNK = 128
NB = 8
GRP = 49
CPT = GRP * NB
EPT = CPT * CHUNK
EP = EPT * NS

BS = 2000
GRID = N // BS


def _sc_mesh():
    return plsc.VectorSubcoreMesh(
        core_axis_name="c", subcore_axis_name="s", num_cores=NC,
        num_subcores=NS)


SB = 112
NSB = RPT // SB


def _zero_wbuf(wbuf, W):
    offs = list(range(0, W - 15, 16))
    if W % 16:
        offs.append(W - 16)

    def zrow(r, carry):
        for o in offs:
            wbuf[r, pl.ds(o, 16)] = jnp.zeros((16,), jnp.float32)
        return carry
    lax.fori_loop(0, SB, zrow, 0)


def _zero_slab_stripe(wbuf, slab, tid):
    def zcp(k, carry):
        off = tid * RPT + k * SB
        pltpu.sync_copy(wbuf, slab.at[pl.ds(off, SB)])
        return carry
    lax.fori_loop(0, NSB, zcp, 0)


def _writeout_stripe(wbuf, slab, out_hbm, cid, tid):
    def wcp(k, carry):
        off = tid * RPT + k * SB
        pltpu.sync_copy(slab.at[pl.ds(off, SB)], wbuf)
        pltpu.sync_copy(wbuf, out_hbm.at[pl.ds(cid * RS + off, SB)])
        return carry
    lax.fori_loop(0, NSB, wcp, 0)


def _local_dst_group(dstbuf, base_node):
    for j in range(NB * CHUNK // 16):
        d = dstbuf[pl.ds(j * 16, 16)]
        ld = d - base_node
        ok = (ld >= 0) & (ld < NPC)
        dstbuf[pl.ds(j * 16, 16)] = jnp.where(ok, ld, NPC)


def _hop_body(W, v_hbm, src_hbm, dst_hbm, out_hbm,
              srcbuf, dstbuf, rowbufs, wbuf, slab,
              isem, gsem, ssem):
    cid = lax.axis_index("c")
    tid = lax.axis_index("s")
    base_node = cid * NPC
    _zero_wbuf(wbuf, W)
    _zero_slab_stripe(wbuf, slab, tid)
    plsc.subcore_barrier()

    base_row = tid * CPT

    def group(g, carry):
        eb = (base_row + g * NB) * CHUNK
        ia = pltpu.async_copy(src_hbm.at[pl.ds(eb, NB * CHUNK)], srcbuf, isem)
        ib = pltpu.async_copy(dst_hbm.at[pl.ds(eb, NB * CHUNK)], dstbuf, isem)
        ia.wait()
        ib.wait()
        _local_dst_group(dstbuf, base_node)
        pltpu.async_copy(v_hbm.at[srcbuf], rowbufs, gsem).wait()
        pltpu.async_copy(rowbufs, slab.at[dstbuf], ssem, add=True).wait()
        return carry

    lax.fori_loop(0, GRP, group, 0)
    plsc.subcore_barrier()
    _writeout_stripe(wbuf, slab, out_hbm, cid, tid)


def _make_hop(W):
    body = functools.partial(_hop_body, W)
    return pl.kernel(
        body,
        out_type=jax.ShapeDtypeStruct((NC * RS, W), jnp.float32),
        mesh=_sc_mesh(),
        compiler_params=pltpu.CompilerParams(use_tc_tiling_on_sc=False),
        scratch_types=[
            pltpu.VMEM((NB * CHUNK,), jnp.int32),
            pltpu.VMEM((NB * CHUNK,), jnp.int32),
            pltpu.VMEM((NB * CHUNK, W), jnp.float32),
            pltpu.VMEM((SB, W), jnp.float32),
            pltpu.VMEM_SHARED((RS, W), jnp.float32),
            pltpu.SemaphoreType.DMA,
            pltpu.SemaphoreType.DMA,
            pltpu.SemaphoreType.DMA,
        ],
    )


def _deg_body(dst_hbm, out_hbm, dstbuf, onesbuf, wbuf, slab,
              isem, ssem):
    cid = lax.axis_index("c")
    tid = lax.axis_index("s")
    base_node = cid * NPC
    _zero_wbuf(wbuf, W2P)

    def orow(r, carry):
        onesbuf[r, pl.ds(0, 16)] = jnp.ones((16,), jnp.float32)
        return carry
    lax.fori_loop(0, NB * CHUNK, orow, 0)
    _zero_slab_stripe(wbuf, slab, tid)
    plsc.subcore_barrier()

    base_row = tid * CPT

    def group(g, carry):
        eb = (base_row + g * NB) * CHUNK
        pltpu.async_copy(dst_hbm.at[pl.ds(eb, NB * CHUNK)], dstbuf, isem).wait()
        _local_dst_group(dstbuf, base_node)
        pltpu.async_copy(onesbuf, slab.at[dstbuf], ssem, add=True).wait()
        return carry

    lax.fori_loop(0, GRP, group, 0)
    plsc.subcore_barrier()
    _writeout_stripe(wbuf, slab, out_hbm, cid, tid)


def _make_deg():
    return pl.kernel(
        _deg_body,
        out_type=jax.ShapeDtypeStruct((NC * RS, W2P), jnp.float32),
        mesh=_sc_mesh(),
        compiler_params=pltpu.CompilerParams(use_tc_tiling_on_sc=False),
        scratch_types=[
            pltpu.VMEM((NB * CHUNK,), jnp.int32),
            pltpu.VMEM((NB * CHUNK, W2P), jnp.float32),
            pltpu.VMEM((SB, W2P), jnp.float32),
            pltpu.VMEM_SHARED((RS, W2P), jnp.float32),
            pltpu.SemaphoreType.DMA,
            pltpu.SemaphoreType.DMA,
        ],
    )


def _unpad_core(a):
    return jnp.concatenate([a[:NPC], a[RS:RS + NPC]], axis=0)


def _stats_body(x_ref, sum_ref, sumsq_ref):
    i = pl.program_id(0)

    @pl.when(i == 0)
    def _():
        sum_ref[...] = jnp.zeros_like(sum_ref)
        sumsq_ref[...] = jnp.zeros_like(sumsq_ref)

    xb = x_ref[...]
    F = xb.shape[-1]
    xr = xb.reshape(BS // 8, 8, F)
    sum_ref[...] += jnp.sum(xr, axis=0)
    sumsq_ref[...] += jnp.sum(xr * xr, axis=0)


def _stats(x):
    F = x.shape[-1]
    return pl.pallas_call(
        _stats_body,
        grid=(GRID,),
        in_specs=[pl.BlockSpec((BS, F), lambda i: (i, 0))],
        out_specs=[pl.BlockSpec((8, F), lambda i: (0, 0)),
                   pl.BlockSpec((8, F), lambda i: (0, 0))],
        out_shape=[jax.ShapeDtypeStruct((8, F), jnp.float32),
                   jax.ShapeDtypeStruct((8, F), jnp.float32)],
    )(x)


def _apply1_body(x_ref, sum_ref, sumsq_ref, g_ref, b_ref, deg_ref,
                 h0_ref, hta_ref, htb_ref, dis_ref, invdeg_ref):
    mu = jnp.sum(sum_ref[...], axis=0, keepdims=True) / N
    var = jnp.sum(sumsq_ref[...], axis=0, keepdims=True) / N - mu * mu
    rstd = lax.rsqrt(var + EPS)
    h0 = (x_ref[...] - mu) * rstd * g_ref[...] + b_ref[...]
    d = deg_ref[...]
    dis = jnp.where(d > 0, lax.rsqrt(jnp.maximum(d, 1e-12)), 0.0)
    h0_ref[...] = h0
    ht = dis * h0
    hta_ref[...] = ht[:, :WH]
    htb_ref[...] = ht[:, WH:]
    dis_ref[...] = dis
    invdeg_ref[...] = dis * dis


def _apply1(x, sums, sumsqs, gamma, beta, deg):
    return pl.pallas_call(
        _apply1_body,
        grid=(GRID,),
        in_specs=[pl.BlockSpec((BS, WP), lambda i: (i, 0)),
                  pl.BlockSpec((8, WP), lambda i: (0, 0)),
                  pl.BlockSpec((8, WP), lambda i: (0, 0)),
                  pl.BlockSpec((1, WP), lambda i: (0, 0)),
                  pl.BlockSpec((1, WP), lambda i: (0, 0)),
                  pl.BlockSpec((BS, 1), lambda i: (i, 0))],
        out_specs=[pl.BlockSpec((BS, WP), lambda i: (i, 0)),
                   pl.BlockSpec((BS, WH), lambda i: (i, 0)),
                   pl.BlockSpec((BS, WH), lambda i: (i, 0)),
                   pl.BlockSpec((BS, 1), lambda i: (i, 0)),
                   pl.BlockSpec((BS, 1), lambda i: (i, 0))],
        out_shape=[jax.ShapeDtypeStruct((N, WP), jnp.float32),
                   jax.ShapeDtypeStruct((N, WH), jnp.float32),
                   jax.ShapeDtypeStruct((N, WH), jnp.float32),
                   jax.ShapeDtypeStruct((N, 1), jnp.float32),
                   jax.ShapeDtypeStruct((N, 1), jnp.float32)],
    )(x, sums, sumsqs, gamma, beta, deg)


def _scale_body(n_in, *refs):
    g_refs = refs[:n_in]
    invdeg_ref = refs[n_in]
    v_refs = refs[n_in + 1:]
    invdeg = invdeg_ref[...]
    for k in range(n_in):
        v_refs[k][...] = invdeg * g_refs[k][...]


def _scale(gs, invdeg):
    n_in = len(gs)
    blk = [pl.BlockSpec((BS, g.shape[-1]), lambda i: (i, 0)) for g in gs]
    return pl.pallas_call(
        functools.partial(_scale_body, n_in),
        grid=(GRID,),
        in_specs=blk + [pl.BlockSpec((BS, 1), lambda i: (i, 0))],
        out_specs=blk,
        out_shape=[jax.ShapeDtypeStruct((N, g.shape[-1]), jnp.float32)
                   for g in gs],
    )(*gs, invdeg)


def _mm1_body(h0_ref, p1a_ref, p1b_ref, p2a_ref, p2b_ref, p3a_ref, p3b_ref,
              dis_ref, w0_ref, wa_ref, wb_ref, b_ref,
              out_ref, sum_ref, sumsq_ref):
    i = pl.program_id(0)
    f32 = jnp.float32
    dis = dis_ref[...]
    acc = jnp.dot(h0_ref[...], w0_ref[...], preferred_element_type=f32)
    acc += jnp.dot(dis * p1a_ref[...], wa_ref[0], preferred_element_type=f32)
    acc += jnp.dot(dis * p1b_ref[...], wb_ref[0], preferred_element_type=f32)
    acc += jnp.dot(dis * p2a_ref[...], wa_ref[1], preferred_element_type=f32)
    acc += jnp.dot(dis * p2b_ref[...], wb_ref[1], preferred_element_type=f32)
    acc += jnp.dot(dis * p3a_ref[...], wa_ref[2], preferred_element_type=f32)
    acc += jnp.dot(dis * p3b_ref[...], wb_ref[2], preferred_element_type=f32)
    acc += b_ref[...]
    out_ref[...] = acc

    @pl.when(i == 0)
    def _():
        sum_ref[...] = jnp.zeros_like(sum_ref)
        sumsq_ref[...] = jnp.zeros_like(sumsq_ref)

    ar = acc.reshape(BS // 8, 8, DH)
    sum_ref[...] += jnp.sum(ar, axis=0)
    sumsq_ref[...] += jnp.sum(ar * ar, axis=0)


def _mm1(h0, ps, dis, w0, wa, wb, b):
    phalf = pl.BlockSpec((BS, WH), lambda i: (i, 0))
    return pl.pallas_call(
        _mm1_body,
        grid=(GRID,),
        in_specs=[pl.BlockSpec((BS, WP), lambda i: (i, 0))]
                 + [phalf] * 6
                 + [pl.BlockSpec((BS, 1), lambda i: (i, 0)),
                    pl.BlockSpec((WP, DH), lambda i: (0, 0)),
                    pl.BlockSpec((3, WH, DH), lambda i: (0, 0, 0)),
                    pl.BlockSpec((3, WH, DH), lambda i: (0, 0, 0)),
                    pl.BlockSpec((1, DH), lambda i: (0, 0))],
        out_specs=[pl.BlockSpec((BS, DH), lambda i: (i, 0)),
                   pl.BlockSpec((8, DH), lambda i: (0, 0)),
                   pl.BlockSpec((8, DH), lambda i: (0, 0))],
        out_shape=[jax.ShapeDtypeStruct((N, DH), jnp.float32),
                   jax.ShapeDtypeStruct((8, DH), jnp.float32),
                   jax.ShapeDtypeStruct((8, DH), jnp.float32)],
    )(h0, *ps, dis, w0, wa, wb, b)


def _z_body(o_ref, sum_ref, sumsq_ref, g_ref, b_ref, wz_ref, dis_ref,
            z0_ref, u0_ref):
    mu = jnp.sum(sum_ref[...], axis=0, keepdims=True) / N
    var = jnp.sum(sumsq_ref[...], axis=0, keepdims=True) / N - mu * mu
    rstd = lax.rsqrt(var + EPS)
    h2 = (o_ref[...] - mu) * rstd * g_ref[...] + b_ref[...]
    h2 = jnp.where(h2 > 0, h2, SLOPE * h2)
    zf = jnp.dot(h2, wz_ref[...], preferred_element_type=jnp.float32)
    z0_ref[...] = zf[:, 0:1]
    mask = (lax.broadcasted_iota(jnp.int32, (1, W2P), 1) >= 1) & (
        lax.broadcasted_iota(jnp.int32, (1, W2P), 1) <= 3)
    u0_ref[...] = jnp.where(mask, dis_ref[...] * zf, 0.0)


def _zproj(out1, sums, sumsqs, gamma, beta, wz, dis):
    return pl.pallas_call(
        _z_body,
        grid=(GRID,),
        in_specs=[pl.BlockSpec((BS, DH), lambda i: (i, 0)),
                  pl.BlockSpec((8, DH), lambda i: (0, 0)),
                  pl.BlockSpec((8, DH), lambda i: (0, 0)),
                  pl.BlockSpec((1, DH), lambda i: (0, 0)),
                  pl.BlockSpec((1, DH), lambda i: (0, 0)),
                  pl.BlockSpec((DH, W2P), lambda i: (0, 0)),
                  pl.BlockSpec((BS, 1), lambda i: (i, 0))],
        out_specs=[pl.BlockSpec((BS, 1), lambda i: (i, 0)),
                   pl.BlockSpec((BS, W2P), lambda i: (i, 0))],
        out_shape=[jax.ShapeDtypeStruct((N, 1), jnp.float32),
                   jax.ShapeDtypeStruct((N, W2P), jnp.float32)],
    )(out1, sums, sumsqs, gamma, beta, wz, dis)


def _final_body(z0_ref, g1_ref, g2_ref, g3_ref, dis_ref, b_ref, out_ref):
    dis = dis_ref[...]
    out_ref[...] = (z0_ref[...] + dis * g1_ref[:, 1:2] + dis * g2_ref[:, 2:3]
                    + dis * g3_ref[:, 3:4] + b_ref[...])


def _final(z0, g1, g2, g3, dis, b2):
    return pl.pallas_call(
        _final_body,
        grid=(GRID,),
        in_specs=[pl.BlockSpec((BS, 1), lambda i: (i, 0)),
                  pl.BlockSpec((BS, W2P), lambda i: (i, 0)),
                  pl.BlockSpec((BS, W2P), lambda i: (i, 0)),
                  pl.BlockSpec((BS, W2P), lambda i: (i, 0)),
                  pl.BlockSpec((BS, 1), lambda i: (i, 0)),
                  pl.BlockSpec((1, 1), lambda i: (0, 0))],
        out_specs=pl.BlockSpec((BS, 1), lambda i: (i, 0)),
        out_shape=jax.ShapeDtypeStruct((N, 1), jnp.float32),
    )(z0, g1, g2, g3, dis, b2)


def kernel(x, edge_index, gamma1, beta1, W1, b1, gamma2, beta2, W2, b2):
    f32 = jnp.float32
    src = jnp.pad(edge_index[0].astype(jnp.int32), (0, EP - E))
    dst = jnp.pad(edge_index[1].astype(jnp.int32), (0, EP - E),
                  constant_values=-1)

    x_pad = jnp.pad(x, ((0, 0), (0, WP - DIN)))
    g1 = jnp.pad(gamma1, (0, WP - DIN)).reshape(1, WP)
    be1 = jnp.pad(beta1, (0, WP - DIN)).reshape(1, WP)
    W1p = jnp.pad(W1, ((0, 0), (0, WP - DIN), (0, 0)))
    w0 = W1p[0]
    wa = W1p[1:, :WH, :]
    wb = W1p[1:, WH:, :]
    b1r = b1.reshape(1, DH)
    g2 = gamma2.reshape(1, DH)
    be2 = beta2.reshape(1, DH)
    wz = jnp.pad(jnp.transpose(W2[:, :, 0], (1, 0)), ((0, 0), (0, W2P - 4)))
    b2r = b2.reshape(1, 1)

    hop40 = _make_hop(WH)
    hop16 = _make_hop(W2P)
    degk = _make_deg()

    deg = _unpad_core(degk(dst))[:, 0:1]
    sums1, sumsqs1 = _stats(x_pad)
    h0, hta, htb, dis, invdeg = _apply1(x_pad, sums1, sumsqs1, g1, be1, deg)

    def hop_pair(va, vb):
        return (_unpad_core(hop40(va, src, dst)),
                _unpad_core(hop40(vb, src, dst)))

    g1a, g1b = hop_pair(hta, htb)
    v1a, v1b = _scale([g1a, g1b], invdeg)
    g2a, g2b = hop_pair(v1a, v1b)
    v2a, v2b = _scale([g2a, g2b], invdeg)
    g3a, g3b = hop_pair(v2a, v2b)

    out1, sums2, sumsqs2 = _mm1(
        h0, (g1a, g1b, g2a, g2b, g3a, g3b), dis, w0, wa, wb, b1r)
    z0, u0 = _zproj(out1, sums2, sumsqs2, g2, be2, wz, dis)

    ha = _unpad_core(hop16(u0, src, dst))
    (w1v,) = _scale([ha], invdeg)
    hb = _unpad_core(hop16(w1v, src, dst))
    (w2v,) = _scale([hb], invdeg)
    hc = _unpad_core(hop16(w2v, src, dst))

    return _final(z0, ha, hb, hc, dis, b2r)


# speedup vs baseline: 1.0986x; 1.0758x over previous
import functools

import jax
import jax.numpy as jnp
from jax import lax
from jax.experimental import pallas as pl
from jax.experimental.pallas import tpu as pltpu
from jax.experimental.pallas import tpu_sc as plsc

N = 50000
E = 800000
DIN = 67
DH = 128
WP = 80
WH = 40
W2P = 16
EPS = 1e-5
SLOPE = 0.01

NC = 2
NS = 16
NPC = N // NC
RPT = 1568
RS = RPT * NS
CHUNK = 128
NB = 8
GRP = 49
CPT = GRP * NB
EPT = CPT * CHUNK
EP = EPT * NS

BS = 2000
GRID = N // BS


def _sc_mesh():
    return plsc.VectorSubcoreMesh(
        core_axis_name="c", subcore_axis_name="s", num_cores=NC,
        num_subcores=NS)


SB = 112
NSB = RPT // SB


def _zero_wbuf(wbuf, W):
    offs = list(range(0, W - 15, 16))
    if W % 16:
        offs.append(W - 16)

    def zrow(r, carry):
        for o in offs:
            wbuf[r, pl.ds(o, 16)] = jnp.zeros((16,), jnp.float32)
        return carry
    lax.fori_loop(0, SB, zrow, 0)


def _zero_slab_stripe(wbuf, slab, tid):
    def zcp(k, carry):
        off = tid * RPT + k * SB
        pltpu.sync_copy(wbuf, slab.at[pl.ds(off, SB)])
        return carry
    lax.fori_loop(0, NSB, zcp, 0)


def _writeout_stripe(wbuf, slab, out_hbm, cid, tid):
    def wcp(k, carry):
        off = tid * RPT + k * SB
        pltpu.sync_copy(slab.at[pl.ds(off, SB)], wbuf)
        pltpu.sync_copy(wbuf, out_hbm.at[pl.ds(cid * RS + off, SB)])
        return carry
    lax.fori_loop(0, NSB, wcp, 0)


def _local_dst_group(dstbuf, base_node):
    for j in range(NB * CHUNK // 16):
        d = dstbuf[pl.ds(j * 16, 16)]
        ld = d - base_node
        ok = (ld >= 0) & (ld < NPC)
        dstbuf[pl.ds(j * 16, 16)] = jnp.where(ok, ld, NPC)


def _hop_body(W, v_hbm, src_hbm, dst_hbm, out_hbm,
              srcbuf, dstbuf, rowbufs, wbuf, slab,
              isem, gsem, ssem):
    cid = lax.axis_index("c")
    tid = lax.axis_index("s")
    base_node = cid * NPC
    _zero_wbuf(wbuf, W)
    _zero_slab_stripe(wbuf, slab, tid)
    plsc.subcore_barrier()

    base_row = tid * CPT

    def group(g, carry):
        eb = (base_row + g * NB) * CHUNK
        ia = pltpu.async_copy(src_hbm.at[pl.ds(eb, NB * CHUNK)], srcbuf, isem)
        ib = pltpu.async_copy(dst_hbm.at[pl.ds(eb, NB * CHUNK)], dstbuf, isem)
        ia.wait()
        ib.wait()
        _local_dst_group(dstbuf, base_node)
        pltpu.async_copy(v_hbm.at[srcbuf], rowbufs, gsem).wait()
        pltpu.async_copy(rowbufs, slab.at[dstbuf], ssem, add=True).wait()
        return carry

    lax.fori_loop(0, GRP, group, 0)
    plsc.subcore_barrier()
    _writeout_stripe(wbuf, slab, out_hbm, cid, tid)


def _make_hop(W):
    body = functools.partial(_hop_body, W)
    return pl.kernel(
        body,
        out_type=jax.ShapeDtypeStruct((NC * RS, W), jnp.float32),
        mesh=_sc_mesh(),
        compiler_params=pltpu.CompilerParams(use_tc_tiling_on_sc=False),
        scratch_types=[
            pltpu.VMEM((NB * CHUNK,), jnp.int32),
            pltpu.VMEM((NB * CHUNK,), jnp.int32),
            pltpu.VMEM((NB * CHUNK, W), jnp.float32),
            pltpu.VMEM((SB, W), jnp.float32),
            pltpu.VMEM_SHARED((RS, W), jnp.float32),
            pltpu.SemaphoreType.DMA,
            pltpu.SemaphoreType.DMA,
            pltpu.SemaphoreType.DMA,
        ],
    )


def _deg_body(dst_hbm, out_hbm, dstbuf, hist, red, dout, stage, isem):
    cid = lax.axis_index("c")
    tid = lax.axis_index("s")
    base_node = cid * NPC

    def zh(i, carry):
        hist[pl.ds(i * 16, 16)] = jnp.zeros((16,), jnp.float32)
        return carry
    lax.fori_loop(0, RS // 16, zh, 0)

    ones16 = jnp.ones((16,), jnp.float32)
    base_row = tid * CPT

    def group(g, carry):
        eb = (base_row + g * NB) * CHUNK
        pltpu.async_copy(dst_hbm.at[pl.ds(eb, NB * CHUNK)], dstbuf,
                         isem).wait()
        for j in range(NB * CHUNK // 16):
            d = dstbuf[pl.ds(j * 16, 16)]
            ld = d - base_node
            ok = (ld >= 0) & (ld < NPC)
            lds = jnp.where(ok, ld, 0)
            plsc.addupdate_scatter(hist, [lds], ones16, mask=ok)
        return carry

    lax.fori_loop(0, GRP, group, 0)
    pltpu.sync_copy(hist, stage.at[tid])
    plsc.subcore_barrier()
    pltpu.sync_copy(stage.at[:, pl.ds(tid * RPT, RPT)], red)

    def rj(j, carry):
        acc = red[0, pl.ds(j * 16, 16)]
        for k in range(1, NS):
            acc = acc + red[k, pl.ds(j * 16, 16)]
        dout[pl.ds(j * 16, 16)] = acc
        return carry
    lax.fori_loop(0, RPT // 16, rj, 0)
    pltpu.sync_copy(dout, out_hbm.at[pl.ds(cid * RS + tid * RPT, RPT)])


def _make_deg():
    return pl.kernel(
        _deg_body,
        out_type=jax.ShapeDtypeStruct((NC * RS,), jnp.float32),
        mesh=_sc_mesh(),
        compiler_params=pltpu.CompilerParams(
            use_tc_tiling_on_sc=False, needs_layout_passes=False),
        scratch_types=[
            pltpu.VMEM((NB * CHUNK,), jnp.int32),
            pltpu.VMEM((RS,), jnp.float32),
            pltpu.VMEM((NS, RPT), jnp.float32),
            pltpu.VMEM((RPT,), jnp.float32),
            pltpu.VMEM_SHARED((NS, RS), jnp.float32),
            pltpu.SemaphoreType.DMA,
        ],
    )


def _unpad_core(a):
    return jnp.concatenate([a[:NPC], a[RS:RS + NPC]], axis=0)


def _stats_body(x_ref, sum_ref, sumsq_ref):
    i = pl.program_id(0)

    @pl.when(i == 0)
    def _():
        sum_ref[...] = jnp.zeros_like(sum_ref)
        sumsq_ref[...] = jnp.zeros_like(sumsq_ref)

    xb = x_ref[...]
    F = xb.shape[-1]
    xr = xb.reshape(BS // 8, 8, F)
    sum_ref[...] += jnp.sum(xr, axis=0)
    sumsq_ref[...] += jnp.sum(xr * xr, axis=0)


def _stats(x):
    F = x.shape[-1]
    return pl.pallas_call(
        _stats_body,
        grid=(GRID,),
        in_specs=[pl.BlockSpec((BS, F), lambda i: (i, 0))],
        out_specs=[pl.BlockSpec((8, F), lambda i: (0, 0)),
                   pl.BlockSpec((8, F), lambda i: (0, 0))],
        out_shape=[jax.ShapeDtypeStruct((8, F), jnp.float32),
                   jax.ShapeDtypeStruct((8, F), jnp.float32)],
    )(x)


def _apply1_body(x_ref, sum_ref, sumsq_ref, g_ref, b_ref, deg_ref,
                 h0_ref, hta_ref, htb_ref, dis_ref, invdeg_ref):
    mu = jnp.sum(sum_ref[...], axis=0, keepdims=True) / N
    var = jnp.sum(sumsq_ref[...], axis=0, keepdims=True) / N - mu * mu
    rstd = lax.rsqrt(var + EPS)
    h0 = (x_ref[...] - mu) * rstd * g_ref[...] + b_ref[...]
    d = deg_ref[...]
    dis = jnp.where(d > 0, lax.rsqrt(jnp.maximum(d, 1e-12)), 0.0)
    h0_ref[...] = h0
    ht = dis * h0
    hta_ref[...] = ht[:, :WH]
    htb_ref[...] = ht[:, WH:]
    dis_ref[...] = dis
    invdeg_ref[...] = dis * dis


def _apply1(x, sums, sumsqs, gamma, beta, deg):
    return pl.pallas_call(
        _apply1_body,
        grid=(GRID,),
        in_specs=[pl.BlockSpec((BS, WP), lambda i: (i, 0)),
                  pl.BlockSpec((8, WP), lambda i: (0, 0)),
                  pl.BlockSpec((8, WP), lambda i: (0, 0)),
                  pl.BlockSpec((1, WP), lambda i: (0, 0)),
                  pl.BlockSpec((1, WP), lambda i: (0, 0)),
                  pl.BlockSpec((BS, 1), lambda i: (i, 0))],
        out_specs=[pl.BlockSpec((BS, WP), lambda i: (i, 0)),
                   pl.BlockSpec((BS, WH), lambda i: (i, 0)),
                   pl.BlockSpec((BS, WH), lambda i: (i, 0)),
                   pl.BlockSpec((BS, 1), lambda i: (i, 0)),
                   pl.BlockSpec((BS, 1), lambda i: (i, 0))],
        out_shape=[jax.ShapeDtypeStruct((N, WP), jnp.float32),
                   jax.ShapeDtypeStruct((N, WH), jnp.float32),
                   jax.ShapeDtypeStruct((N, WH), jnp.float32),
                   jax.ShapeDtypeStruct((N, 1), jnp.float32),
                   jax.ShapeDtypeStruct((N, 1), jnp.float32)],
    )(x, sums, sumsqs, gamma, beta, deg)


def _scale_body(n_in, *refs):
    g_refs = refs[:n_in]
    invdeg_ref = refs[n_in]
    v_refs = refs[n_in + 1:]
    invdeg = invdeg_ref[...]
    for k in range(n_in):
        v_refs[k][...] = invdeg * g_refs[k][...]


def _scale(gs, invdeg):
    n_in = len(gs)
    blk = [pl.BlockSpec((BS, g.shape[-1]), lambda i: (i, 0)) for g in gs]
    return pl.pallas_call(
        functools.partial(_scale_body, n_in),
        grid=(GRID,),
        in_specs=blk + [pl.BlockSpec((BS, 1), lambda i: (i, 0))],
        out_specs=blk,
        out_shape=[jax.ShapeDtypeStruct((N, g.shape[-1]), jnp.float32)
                   for g in gs],
    )(*gs, invdeg)


def _mm1_body(h0_ref, p1a_ref, p1b_ref, p2a_ref, p2b_ref, p3a_ref, p3b_ref,
              dis_ref, w0_ref, wa_ref, wb_ref, b_ref,
              out_ref, sum_ref, sumsq_ref):
    i = pl.program_id(0)
    f32 = jnp.float32
    dis = dis_ref[...]
    acc = jnp.dot(h0_ref[...], w0_ref[...], preferred_element_type=f32)
    acc += jnp.dot(dis * p1a_ref[...], wa_ref[0], preferred_element_type=f32)
    acc += jnp.dot(dis * p1b_ref[...], wb_ref[0], preferred_element_type=f32)
    acc += jnp.dot(dis * p2a_ref[...], wa_ref[1], preferred_element_type=f32)
    acc += jnp.dot(dis * p2b_ref[...], wb_ref[1], preferred_element_type=f32)
    acc += jnp.dot(dis * p3a_ref[...], wa_ref[2], preferred_element_type=f32)
    acc += jnp.dot(dis * p3b_ref[...], wb_ref[2], preferred_element_type=f32)
    acc += b_ref[...]
    out_ref[...] = acc

    @pl.when(i == 0)
    def _():
        sum_ref[...] = jnp.zeros_like(sum_ref)
        sumsq_ref[...] = jnp.zeros_like(sumsq_ref)

    ar = acc.reshape(BS // 8, 8, DH)
    sum_ref[...] += jnp.sum(ar, axis=0)
    sumsq_ref[...] += jnp.sum(ar * ar, axis=0)


def _mm1(h0, ps, dis, w0, wa, wb, b):
    phalf = pl.BlockSpec((BS, WH), lambda i: (i, 0))
    return pl.pallas_call(
        _mm1_body,
        grid=(GRID,),
        in_specs=[pl.BlockSpec((BS, WP), lambda i: (i, 0))]
                 + [phalf] * 6
                 + [pl.BlockSpec((BS, 1), lambda i: (i, 0)),
                    pl.BlockSpec((WP, DH), lambda i: (0, 0)),
                    pl.BlockSpec((3, WH, DH), lambda i: (0, 0, 0)),
                    pl.BlockSpec((3, WH, DH), lambda i: (0, 0, 0)),
                    pl.BlockSpec((1, DH), lambda i: (0, 0))],
        out_specs=[pl.BlockSpec((BS, DH), lambda i: (i, 0)),
                   pl.BlockSpec((8, DH), lambda i: (0, 0)),
                   pl.BlockSpec((8, DH), lambda i: (0, 0))],
        out_shape=[jax.ShapeDtypeStruct((N, DH), jnp.float32),
                   jax.ShapeDtypeStruct((8, DH), jnp.float32),
                   jax.ShapeDtypeStruct((8, DH), jnp.float32)],
    )(h0, *ps, dis, w0, wa, wb, b)


def _z_body(o_ref, sum_ref, sumsq_ref, g_ref, b_ref, wz_ref, dis_ref,
            z0_ref, u0_ref):
    mu = jnp.sum(sum_ref[...], axis=0, keepdims=True) / N
    var = jnp.sum(sumsq_ref[...], axis=0, keepdims=True) / N - mu * mu
    rstd = lax.rsqrt(var + EPS)
    h2 = (o_ref[...] - mu) * rstd * g_ref[...] + b_ref[...]
    h2 = jnp.where(h2 > 0, h2, SLOPE * h2)
    zf = jnp.dot(h2, wz_ref[...], preferred_element_type=jnp.float32)
    z0_ref[...] = zf[:, 0:1]
    mask = (lax.broadcasted_iota(jnp.int32, (1, W2P), 1) >= 1) & (
        lax.broadcasted_iota(jnp.int32, (1, W2P), 1) <= 3)
    u0_ref[...] = jnp.where(mask, dis_ref[...] * zf, 0.0)


def _zproj(out1, sums, sumsqs, gamma, beta, wz, dis):
    return pl.pallas_call(
        _z_body,
        grid=(GRID,),
        in_specs=[pl.BlockSpec((BS, DH), lambda i: (i, 0)),
                  pl.BlockSpec((8, DH), lambda i: (0, 0)),
                  pl.BlockSpec((8, DH), lambda i: (0, 0)),
                  pl.BlockSpec((1, DH), lambda i: (0, 0)),
                  pl.BlockSpec((1, DH), lambda i: (0, 0)),
                  pl.BlockSpec((DH, W2P), lambda i: (0, 0)),
                  pl.BlockSpec((BS, 1), lambda i: (i, 0))],
        out_specs=[pl.BlockSpec((BS, 1), lambda i: (i, 0)),
                   pl.BlockSpec((BS, W2P), lambda i: (i, 0))],
        out_shape=[jax.ShapeDtypeStruct((N, 1), jnp.float32),
                   jax.ShapeDtypeStruct((N, W2P), jnp.float32)],
    )(out1, sums, sumsqs, gamma, beta, wz, dis)


def _final_body(z0_ref, g1_ref, g2_ref, g3_ref, dis_ref, b_ref, out_ref):
    dis = dis_ref[...]
    out_ref[...] = (z0_ref[...] + dis * g1_ref[:, 1:2] + dis * g2_ref[:, 2:3]
                    + dis * g3_ref[:, 3:4] + b_ref[...])


def _final(z0, g1, g2, g3, dis, b2):
    return pl.pallas_call(
        _final_body,
        grid=(GRID,),
        in_specs=[pl.BlockSpec((BS, 1), lambda i: (i, 0)),
                  pl.BlockSpec((BS, W2P), lambda i: (i, 0)),
                  pl.BlockSpec((BS, W2P), lambda i: (i, 0)),
                  pl.BlockSpec((BS, W2P), lambda i: (i, 0)),
                  pl.BlockSpec((BS, 1), lambda i: (i, 0)),
                  pl.BlockSpec((1, 1), lambda i: (0, 0))],
        out_specs=pl.BlockSpec((BS, 1), lambda i: (i, 0)),
        out_shape=jax.ShapeDtypeStruct((N, 1), jnp.float32),
    )(z0, g1, g2, g3, dis, b2)


def kernel(x, edge_index, gamma1, beta1, W1, b1, gamma2, beta2, W2, b2):
    f32 = jnp.float32
    src = jnp.pad(edge_index[0].astype(jnp.int32), (0, EP - E))
    dst = jnp.pad(edge_index[1].astype(jnp.int32), (0, EP - E),
                  constant_values=-1)

    x_pad = jnp.pad(x, ((0, 0), (0, WP - DIN)))
    g1 = jnp.pad(gamma1, (0, WP - DIN)).reshape(1, WP)
    be1 = jnp.pad(beta1, (0, WP - DIN)).reshape(1, WP)
    W1p = jnp.pad(W1, ((0, 0), (0, WP - DIN), (0, 0)))
    w0 = W1p[0]
    wa = W1p[1:, :WH, :]
    wb = W1p[1:, WH:, :]
    b1r = b1.reshape(1, DH)
    g2 = gamma2.reshape(1, DH)
    be2 = beta2.reshape(1, DH)
    wz = jnp.pad(jnp.transpose(W2[:, :, 0], (1, 0)), ((0, 0), (0, W2P - 4)))
    b2r = b2.reshape(1, 1)

    hop40 = _make_hop(WH)
    hop16 = _make_hop(W2P)
    degk = _make_deg()

    deg = _unpad_core(degk(dst)).reshape(N, 1)
    sums1, sumsqs1 = _stats(x_pad)
    h0, hta, htb, dis, invdeg = _apply1(x_pad, sums1, sumsqs1, g1, be1, deg)

    def hop_pair(va, vb):
        return (_unpad_core(hop40(va, src, dst)),
                _unpad_core(hop40(vb, src, dst)))

    g1a, g1b = hop_pair(hta, htb)
    v1a, v1b = _scale([g1a, g1b], invdeg)
    g2a, g2b = hop_pair(v1a, v1b)
    v2a, v2b = _scale([g2a, g2b], invdeg)
    g3a, g3b = hop_pair(v2a, v2b)

    out1, sums2, sumsqs2 = _mm1(
        h0, (g1a, g1b, g2a, g2b, g3a, g3b), dis, w0, wa, wb, b1r)
    z0, u0 = _zproj(out1, sums2, sumsqs2, g2, be2, wz, dis)

    ha = _unpad_core(hop16(u0, src, dst))
    (w1v,) = _scale([ha], invdeg)
    hb = _unpad_core(hop16(w1v, src, dst))
    (w2v,) = _scale([hb], invdeg)
    hc = _unpad_core(hop16(w2v, src, dst))

    return _final(z0, ha, hb, hc, dis, b2r)
